# Initial kernel scaffold; baseline (speedup 1.0000x reference)
import jax
import jax.numpy as jnp
from jax.experimental import pallas as pl


def kernel(x):
    raise NotImplementedError("write your pallas kernel here")


# speedup vs baseline: 5.1904x; 5.1904x over previous
import functools

import numpy as np
import jax
import jax.numpy as jnp
from jax.experimental import pallas as pl
from jax.experimental.pallas import tpu as pltpu

_DELTA = 0.05
_K = 64
_RB = 64
_CW = 128
_G = 19
_NRUNS = 5
_BIG_I = np.int32(2**30)


def _lex_lt(va, ia, vb, ib):
    return (va < vb) | ((va == vb) & (ia < ib))


def _lane_iota(w):
    return jax.lax.broadcasted_iota(jnp.int32, (1, w), 1)


def _ce_round(v, i, j, keep_min):
    w = v.shape[1]
    lane = _lane_iota(w)
    low = (lane & j) == 0
    pv = jnp.where(low, pltpu.roll(v, w - j, 1), pltpu.roll(v, j, 1))
    pi = jnp.where(low, pltpu.roll(i, w - j, 1), pltpu.roll(i, j, 1))
    self_lt = _lex_lt(v, i, pv, pi)
    keep_self = self_lt == keep_min
    return jnp.where(keep_self, v, pv), jnp.where(keep_self, i, pi)


def _bitonic_sort(v, i, descending=False):
    w = v.shape[1]
    lane = _lane_iota(w)
    k = 2
    while k <= w:
        if k < w:
            asc = (lane & k) == 0
        else:
            asc = jnp.full_like(lane, not descending, dtype=bool)
        j = k // 2
        while j >= 1:
            low = (lane & j) == 0
            v, i = _ce_round(v, i, j, low == asc)
            j //= 2
        k *= 2
    return v, i


def _bitonic_merge(v, i):
    w = v.shape[1]
    lane = _lane_iota(w)
    j = w // 2
    while j >= 1:
        v, i = _ce_round(v, i, j, (lane & j) == 0)
        j //= 2
    return v, i


def _merge_topk(rv, ri, cv, ci):
    cw = cv.shape[1]
    bv = cv[:, cw - _K:]
    bi = ci[:, cw - _K:]
    take_a = _lex_lt(rv, ri, bv, bi)
    dv = jnp.where(take_a, rv, bv)
    di = jnp.where(take_a, ri, bi)
    return _bitonic_merge(dv, di)


def _kernel_body(rstart_ref, nck_ref, xq_ref, xst_ref, oidx_ref, xpool_ref,
                 ov_ref, oi_ref, oc_ref):
    b = pl.program_id(0)
    q0 = xq_ref[:, 0:1]
    q1 = xq_ref[:, 1:2]
    thr = jnp.float32(_DELTA + 1e-8)

    rv = jnp.full((_RB, _K), jnp.inf, jnp.float32)
    ri = jnp.full((_RB, _K), _BIG_I, jnp.int32)
    cnt = jnp.zeros((_RB, _CW), jnp.int32)

    def chunk_step(base, carry):
        rv, ri, cnt = carry
        base = pl.multiple_of(base, _CW)
        c0 = xst_ref[0:1, pl.ds(base, _CW)]
        c1 = xst_ref[1:2, pl.ds(base, _CW)]
        oi = oidx_ref[0:1, pl.ds(base, _CW)]
        dx = q0 - c0
        dy = q1 - c1
        d2 = dx * dx + dy * dy
        r = jnp.sqrt(d2 + 1e-12)
        mask = r <= thr
        cnt = cnt + mask.astype(jnp.int32)
        kv = jnp.where(mask, r, jnp.inf)
        ki = jnp.where(mask, jnp.broadcast_to(oi, (_RB, _CW)),
                       jnp.int32(_BIG_I))
        kv, ki = _bitonic_sort(kv, ki, descending=True)
        rv, ri = _merge_topk(rv, ri, kv, ki)
        return rv, ri, cnt

    carry = (rv, ri, cnt)
    for k in range(_NRUNS):
        s = rstart_ref[b, k]
        nc = nck_ref[b, k]
        carry = jax.lax.fori_loop(
            0, nc, lambda c, cr, s=s: chunk_step(s + c * _CW, cr), carry)
    rv, ri, cnt = carry

    p0 = xpool_ref[0:1, :]
    p1 = xpool_ref[1:2, :]
    dx = q0 - p0
    dy = q1 - p1
    d2 = dx * dx + dy * dy
    rpool = jnp.sqrt(d2 + 1e-12)
    in_r = rpool <= thr
    kv = jnp.full((_RB, _CW), jnp.inf, jnp.float32)
    pool_iota = jax.lax.broadcasted_iota(jnp.int32, (_RB, _CW), 1)
    ki = jnp.where(in_r, jnp.int32(_BIG_I), pool_iota)
    kv, ki = _bitonic_sort(kv, ki, descending=True)
    rv, ri = _merge_topk(rv, ri, kv, ki)

    ov_ref[...] = jnp.where(rv < jnp.inf, -rv, -jnp.inf)
    oi_ref[...] = ri
    oc_ref[...] = jnp.sum(cnt, axis=1, keepdims=True)


def _block_metadata(cids, cell_start):
    nb = cids.shape[0] // _RB
    cblk = cids.reshape(nb, _RB)
    rlo = cblk[:, 0] // _G
    rhi = cblk[:, -1] // _G
    cx = cblk % _G
    cxlo = jnp.min(cx, axis=1)
    cxhi = jnp.max(cx, axis=1)
    rows = rlo[:, None] - 1 + jnp.arange(_NRUNS, dtype=jnp.int32)[None, :]
    valid = (rows >= 0) & (rows < _G) & (rows <= rhi[:, None] + 1)
    clo = jnp.clip(cxlo - 1, 0, _G - 1)[:, None]
    chi = jnp.clip(cxhi + 1, 0, _G - 1)[:, None]
    rows_c = jnp.clip(rows, 0, _G - 1)
    start = cell_start[rows_c * _G + clo]
    end = cell_start[rows_c * _G + chi + 1]
    start = jnp.where(valid, start, 0)
    end = jnp.where(valid, end, 0)
    start = (start // _CW) * _CW
    end = ((end + _CW - 1) // _CW) * _CW
    ss, ee, prev = [], [], jnp.zeros_like(start[:, 0])
    for k in range(_NRUNS):
        s = jnp.maximum(start[:, k], prev)
        e = jnp.maximum(end[:, k], s)
        ss.append(s)
        ee.append(e)
        prev = e
    rstart = jnp.stack(ss, axis=1).astype(jnp.int32)
    nck = ((jnp.stack(ee, axis=1) - rstart) // _CW).astype(jnp.int32)
    return rstart, nck


def kernel(x):
    n = x.shape[0]
    gx = jnp.clip(jnp.floor(x[:, 0] * _G).astype(jnp.int32), 0, _G - 1)
    gy = jnp.clip(jnp.floor(x[:, 1] * _G).astype(jnp.int32), 0, _G - 1)
    cid = gy * _G + gx
    order = jnp.argsort(cid).astype(jnp.int32)
    xs = x[order]
    cids = cid[order]
    cell_start = jnp.searchsorted(
        cids, jnp.arange(_G * _G + 1, dtype=jnp.int32)).astype(jnp.int32)
    rstart, nck = _block_metadata(cids, cell_start)

    xst = xs.T
    oidx = order.reshape(1, n)
    xpool = x[:_CW].T

    nb = n // _RB
    grid_spec = pltpu.PrefetchScalarGridSpec(
        num_scalar_prefetch=2,
        grid=(nb,),
        in_specs=[
            pl.BlockSpec((_RB, 2), lambda i, *_: (i, 0)),
            pl.BlockSpec((2, n), lambda i, *_: (0, 0)),
            pl.BlockSpec((1, n), lambda i, *_: (0, 0)),
            pl.BlockSpec((2, _CW), lambda i, *_: (0, 0)),
        ],
        out_specs=[
            pl.BlockSpec((_RB, _K), lambda i, *_: (i, 0)),
            pl.BlockSpec((_RB, _K), lambda i, *_: (i, 0)),
            pl.BlockSpec((_RB, 1), lambda i, *_: (i, 0)),
        ],
    )
    ov, oi, oc = pl.pallas_call(
        _kernel_body,
        grid_spec=grid_spec,
        out_shape=[
            jax.ShapeDtypeStruct((n, _K), jnp.float32),
            jax.ShapeDtypeStruct((n, _K), jnp.int32),
            jax.ShapeDtypeStruct((n, 1), jnp.int32),
        ],
        compiler_params=pltpu.CompilerParams(
            dimension_semantics=("arbitrary",),
        ),
    )(rstart, nck, xs, xst, oidx, xpool)

    inv = jnp.argsort(order).astype(jnp.int32)
    return ov[inv], oi[inv], oc.reshape(n)[inv]
